# Initial kernel scaffold; baseline (speedup 1.0000x reference)
import functools

import jax
import jax.numpy as jnp
from jax import lax
from jax.experimental import pallas as pl
from jax.experimental.pallas import tpu as pltpu
from jax.experimental.pallas import tpu_sc as plsc

N = 320000
D = 128
S = 4096
CH = 128
NW = 32
NCH = N // CH
FULL_ROUNDS = NCH // NW
EXTRA = NCH - FULL_ROUNDS * NW


def _sc_partial(x, idx):
    mesh = plsc.VectorSubcoreMesh(core_axis_name="c", subcore_axis_name="s")

    @functools.partial(
        pl.kernel,
        out_type=[
            jax.ShapeDtypeStruct((2, S, D), jnp.float32),
            jax.ShapeDtypeStruct((2, S, 16), jnp.float32),
        ],
        mesh=mesh,
        scratch_types=[
            pltpu.VMEM((CH, D), jnp.float32),
            pltpu.VMEM((1, CH), jnp.int32),
            pltpu.VMEM((CH, 16), jnp.float32),
            pltpu.VMEM((64, D), jnp.float32),
            pltpu.VMEM((256, 16), jnp.float32),
            pltpu.VMEM_SHARED((S, D), jnp.float32),
            pltpu.VMEM_SHARED((S, 16), jnp.float32),
        ],
    )
    def k(x_hbm, idx_hbm, pacc_hbm, pcnt_hbm,
          rows_v, idx_v, ones_v, zrow_v, zcnt_v, acc_sh, cnt_sh):
        c = lax.axis_index("c")
        s = lax.axis_index("s")
        w = c * 16 + s
        zero16 = jnp.zeros((16,), jnp.float32)
        one16 = jnp.ones((16,), jnp.float32)

        @pl.loop(0, 64)
        def _(i):
            for t in range(8):
                zrow_v[i, pl.ds(t * 16, 16)] = zero16

        @pl.loop(0, 256)
        def _(i):
            zcnt_v[i, :] = zero16

        @pl.loop(0, CH)
        def _(i):
            ones_v[i, :] = one16

        for t in range(4):
            pltpu.sync_copy(zrow_v, acc_sh.at[pl.ds(s * 256 + t * 64, 64)])
        pltpu.sync_copy(zcnt_v, cnt_sh.at[pl.ds(s * 256, 256)])
        plsc.subcore_barrier()

        def process(cidx):
            base = cidx * CH
            pltpu.sync_copy(idx_hbm.at[pl.ds(base, CH)], idx_v.at[0])
            pltpu.sync_copy(x_hbm.at[pl.ds(base, CH)], rows_v)
            pltpu.sync_copy(rows_v, acc_sh.at[idx_v.at[0]], add=True)
            pltpu.sync_copy(ones_v, cnt_sh.at[idx_v.at[0]], add=True)

        @pl.loop(0, FULL_ROUNDS)
        def _(j):
            process(w + NW * j)

        @pl.when(w < EXTRA)
        def _():
            process(w + NW * FULL_ROUNDS)

        plsc.subcore_barrier()
        pltpu.sync_copy(acc_sh.at[pl.ds(s * 256, 256)],
                        pacc_hbm.at[c, pl.ds(s * 256, 256)])
        pltpu.sync_copy(cnt_sh.at[pl.ds(s * 256, 256)],
                        pcnt_hbm.at[c, pl.ds(s * 256, 256)])

    return k(x, idx)


def _tc_finalize(pacc, pcnt):
    def body(pacc_ref, pcnt_ref, o_ref):
        ssum = pacc_ref[0] + pacc_ref[1]
        cnt = pcnt_ref[0, :, 0:1] + pcnt_ref[1, :, 0:1]
        o_ref[...] = ssum / jnp.maximum(cnt, 1.0)

    return pl.pallas_call(
        body,
        out_shape=jax.ShapeDtypeStruct((S, D), jnp.float32),
    )(pacc, pcnt)


@jax.jit
def kernel(node_embeddings, batch_indices):
    idx = batch_indices.astype(jnp.int32)
    pacc, pcnt = _sc_partial(node_embeddings, idx)
    return _tc_finalize(pacc, pcnt)


# speedup vs baseline: 1.6489x; 1.6489x over previous
import jax
import jax.numpy as jnp
from jax import lax
from jax.experimental import pallas as pl
from jax.experimental.pallas import tpu as pltpu

N = 320000
D = 128
S = 4096
B = 512
W = 128
NB = N // B
NWIN = S // W
T = NB + 2 * NWIN


def _segment_body(bid_ref, vid_ref, fst_ref, vld_ref,
                  rows_ref, ids_ref, o_ref, o2_ref):
    t = pl.program_id(0)
    v = vid_ref[t]
    fstv = fst_ref[t]
    vld = vld_ref[t]

    ids = ids_ref[0, 0, :]
    local = ids - v * W
    iota = lax.broadcasted_iota(jnp.int32, (W, B), 0)
    oh = jnp.logical_and(iota == local[None, :], vld > 0)
    ohb = oh.astype(jnp.bfloat16)

    rows = rows_ref[...]
    hi = rows.astype(jnp.bfloat16)
    lo = (rows - hi.astype(jnp.float32)).astype(jnp.bfloat16)

    dn = (((1,), (0,)), ((), ()))
    contrib = (lax.dot_general(ohb, hi, dn, preferred_element_type=jnp.float32)
               + lax.dot_general(ohb, lo, dn,
                                 preferred_element_type=jnp.float32))
    cnt = lax.dot_general(ohb, jnp.ones((B, D), jnp.bfloat16), dn,
                          preferred_element_type=jnp.float32)

    @pl.when(fstv == 1)
    def _():
        o_ref[...] = contrib
        o2_ref[...] = cnt

    @pl.when(fstv == 0)
    def _():
        o_ref[...] += contrib
        o2_ref[...] += cnt


def _segment_sums(x, idx):
    edges = jnp.arange(NWIN + 1, dtype=jnp.int32) * W
    ws = jnp.searchsorted(idx, edges, side="left").astype(jnp.int32)
    nonempty = ws[1:] > ws[:-1]
    fb = ws[:-1] // B
    lb = jnp.where(nonempty, (ws[1:] - 1) // B, fb)
    nb = jnp.where(nonempty, lb - fb + 1, 1)
    starts = jnp.concatenate([jnp.zeros((1,), jnp.int32),
                              jnp.cumsum(nb).astype(jnp.int32)])
    treal = starts[NWIN]

    t = jnp.arange(T, dtype=jnp.int32)
    v_t = jnp.clip(jnp.searchsorted(starts, t, side="right").astype(jnp.int32)
                   - 1, 0, NWIN - 1)
    o_t = t - starts[v_t]
    block_id = jnp.clip(jnp.where(nonempty[v_t], fb[v_t] + o_t, 0), 0, NB - 1)
    valid = (t < treal).astype(jnp.int32)
    first = jnp.logical_and(o_t == 0, t < treal).astype(jnp.int32)

    ids3 = idx.reshape(NB, 1, B)

    grid_spec = pltpu.PrefetchScalarGridSpec(
        num_scalar_prefetch=4,
        grid=(T,),
        in_specs=[
            pl.BlockSpec((B, D), lambda t, bid, vid, fst, vld: (bid[t], 0)),
            pl.BlockSpec((1, 1, B),
                         lambda t, bid, vid, fst, vld: (bid[t], 0, 0)),
        ],
        out_specs=[
            pl.BlockSpec((W, D), lambda t, bid, vid, fst, vld: (vid[t], 0)),
            pl.BlockSpec((W, D), lambda t, bid, vid, fst, vld: (vid[t], 0)),
        ],
    )
    sums, cnts = pl.pallas_call(
        _segment_body,
        grid_spec=grid_spec,
        out_shape=[jax.ShapeDtypeStruct((S, D), jnp.float32),
                   jax.ShapeDtypeStruct((S, D), jnp.float32)],
    )(block_id, v_t, first, valid, x, ids3)
    return sums, cnts


def _divide_body(s_ref, c_ref, o_ref):
    o_ref[...] = s_ref[...] / jnp.maximum(c_ref[...], 1.0)


def _finalize(sums, cnts):
    return pl.pallas_call(
        _divide_body,
        out_shape=jax.ShapeDtypeStruct((S, D), jnp.float32),
    )(sums, cnts)


@jax.jit
def kernel(node_embeddings, batch_indices):
    idx = batch_indices.astype(jnp.int32)
    sums, cnts = _segment_sums(node_embeddings, idx)
    return _finalize(sums, cnts)


# speedup vs baseline: 3.1926x; 1.9362x over previous
import jax
import jax.numpy as jnp
from jax import lax
from jax.experimental import pallas as pl
from jax.experimental.pallas import tpu as pltpu

N = 320000
D = 128
S = 4096
B = 1280
W = 256
NB = N // B
NWIN = S // W
T = NB + 2 * NWIN
DA = D + 128


def _segment_body(bid_ref, vid_ref, fst_ref, vld_ref,
                  rows_ref, ids_ref, o_ref):
    t = pl.program_id(0)
    v = vid_ref[t]
    fstv = fst_ref[t]
    vld = vld_ref[t]

    ids = ids_ref[0, 0, :]
    local = ids - v * W
    iota = lax.broadcasted_iota(jnp.int32, (W, B), 0)
    oh = jnp.logical_and(iota == local[None, :], vld > 0)
    ohb = oh.astype(jnp.bfloat16)

    rows = rows_ref[...]
    hi = rows.astype(jnp.bfloat16)
    lo = (rows - hi.astype(jnp.float32)).astype(jnp.bfloat16)
    onecol = jnp.concatenate(
        [jnp.ones((B, 1), jnp.bfloat16), jnp.zeros((B, 127), jnp.bfloat16)],
        axis=1)
    hi_aug = jnp.concatenate([hi, onecol], axis=1)

    dn = (((1,), (0,)), ((), ()))
    contrib = lax.dot_general(ohb, hi_aug, dn,
                              preferred_element_type=jnp.float32)
    lo_c = lax.dot_general(ohb, lo, dn, preferred_element_type=jnp.float32)

    @pl.when(fstv == 1)
    def _():
        o_ref[...] = contrib
        o_ref[:, :D] += lo_c

    @pl.when(fstv == 0)
    def _():
        o_ref[...] += contrib
        o_ref[:, :D] += lo_c


def _segment_sums(x, idx):
    edges = jnp.arange(NWIN + 1, dtype=jnp.int32) * W
    ws = jnp.searchsorted(idx, edges, side="left").astype(jnp.int32)
    nonempty = ws[1:] > ws[:-1]
    fb = ws[:-1] // B
    lb = jnp.where(nonempty, (ws[1:] - 1) // B, fb)
    nb = jnp.where(nonempty, lb - fb + 1, 1)
    starts = jnp.concatenate([jnp.zeros((1,), jnp.int32),
                              jnp.cumsum(nb).astype(jnp.int32)])
    treal = starts[NWIN]

    t = jnp.arange(T, dtype=jnp.int32)
    v_t = jnp.clip(jnp.searchsorted(starts, t, side="right").astype(jnp.int32)
                   - 1, 0, NWIN - 1)
    o_t = t - starts[v_t]
    block_id = jnp.clip(jnp.where(nonempty[v_t], fb[v_t] + o_t, 0), 0, NB - 1)
    valid = (t < treal).astype(jnp.int32)
    first = jnp.logical_and(o_t == 0, t < treal).astype(jnp.int32)

    ids3 = idx.reshape(NB, 1, B)

    grid_spec = pltpu.PrefetchScalarGridSpec(
        num_scalar_prefetch=4,
        grid=(T,),
        in_specs=[
            pl.BlockSpec((B, D), lambda t, bid, vid, fst, vld: (bid[t], 0)),
            pl.BlockSpec((1, 1, B),
                         lambda t, bid, vid, fst, vld: (bid[t], 0, 0)),
        ],
        out_specs=[
            pl.BlockSpec((W, DA), lambda t, bid, vid, fst, vld: (vid[t], 0)),
        ],
    )
    (acc,) = pl.pallas_call(
        _segment_body,
        grid_spec=grid_spec,
        out_shape=[jax.ShapeDtypeStruct((S, DA), jnp.float32)],
    )(block_id, v_t, first, valid, x, ids3)
    return acc


def _divide_body(a_ref, o_ref):
    o_ref[...] = a_ref[:, :D] / jnp.maximum(a_ref[:, D:D + 1], 1.0)


def _finalize(acc):
    return pl.pallas_call(
        _divide_body,
        out_shape=jax.ShapeDtypeStruct((S, D), jnp.float32),
    )(acc)


@jax.jit
def kernel(node_embeddings, batch_indices):
    idx = batch_indices.astype(jnp.int32)
    acc = _segment_sums(node_embeddings, idx)
    return _finalize(acc)


# speedup vs baseline: 4.3617x; 1.3662x over previous
import jax
import jax.numpy as jnp
from jax import lax
from jax.experimental import pallas as pl
from jax.experimental.pallas import tpu as pltpu

N = 320000
D = 128
S = 4096
B = 2560
W = 256
NB = N // B
NWIN = S // W
T = NB + 2 * NWIN
DA = D + 128


def _segment_body(bid_ref, vid_ref, fst_ref, vld_ref,
                  rows_ref, ids_ref, o_ref):
    t = pl.program_id(0)
    v = vid_ref[t]
    fstv = fst_ref[t]
    vld = vld_ref[t]

    ids = ids_ref[0, 0, :]
    local = ids - v * W
    iota = lax.broadcasted_iota(jnp.int32, (W, B), 0)
    oh = jnp.logical_and(iota == local[None, :], vld > 0)
    ohb = oh.astype(jnp.bfloat16)

    rows = rows_ref[...]
    hi = rows.astype(jnp.bfloat16)
    lo = (rows - hi.astype(jnp.float32)).astype(jnp.bfloat16)
    onecol = jnp.concatenate(
        [jnp.ones((B, 1), jnp.bfloat16), jnp.zeros((B, 127), jnp.bfloat16)],
        axis=1)
    hi_aug = jnp.concatenate([hi, onecol], axis=1)

    dn = (((1,), (0,)), ((), ()))
    contrib = lax.dot_general(ohb, hi_aug, dn,
                              preferred_element_type=jnp.float32)
    lo_c = lax.dot_general(ohb, lo, dn, preferred_element_type=jnp.float32)

    @pl.when(fstv == 1)
    def _():
        o_ref[...] = contrib
        o_ref[:, :D] += lo_c

    @pl.when(fstv == 0)
    def _():
        o_ref[...] += contrib
        o_ref[:, :D] += lo_c


def _segment_sums(x, idx):
    edges = jnp.arange(NWIN + 1, dtype=jnp.int32) * W
    ws = jnp.searchsorted(idx, edges, side="left").astype(jnp.int32)
    nonempty = ws[1:] > ws[:-1]
    fb = ws[:-1] // B
    lb = jnp.where(nonempty, (ws[1:] - 1) // B, fb)
    nb = jnp.where(nonempty, lb - fb + 1, 1)
    starts = jnp.concatenate([jnp.zeros((1,), jnp.int32),
                              jnp.cumsum(nb).astype(jnp.int32)])
    treal = starts[NWIN]

    t = jnp.arange(T, dtype=jnp.int32)
    v_t = jnp.clip(jnp.searchsorted(starts, t, side="right").astype(jnp.int32)
                   - 1, 0, NWIN - 1)
    o_t = t - starts[v_t]
    block_id = jnp.clip(jnp.where(nonempty[v_t], fb[v_t] + o_t, 0), 0, NB - 1)
    valid = (t < treal).astype(jnp.int32)
    first = jnp.logical_and(o_t == 0, t < treal).astype(jnp.int32)

    ids3 = idx.reshape(NB, 1, B)

    grid_spec = pltpu.PrefetchScalarGridSpec(
        num_scalar_prefetch=4,
        grid=(T,),
        in_specs=[
            pl.BlockSpec((B, D), lambda t, bid, vid, fst, vld: (bid[t], 0)),
            pl.BlockSpec((1, 1, B),
                         lambda t, bid, vid, fst, vld: (bid[t], 0, 0)),
        ],
        out_specs=[
            pl.BlockSpec((W, DA), lambda t, bid, vid, fst, vld: (vid[t], 0)),
        ],
    )
    (acc,) = pl.pallas_call(
        _segment_body,
        grid_spec=grid_spec,
        out_shape=[jax.ShapeDtypeStruct((S, DA), jnp.float32)],
    )(block_id, v_t, first, valid, x, ids3)
    return acc


def _divide_body(a_ref, o_ref):
    o_ref[...] = a_ref[:, :D] / jnp.maximum(a_ref[:, D:D + 1], 1.0)


def _finalize(acc):
    return pl.pallas_call(
        _divide_body,
        out_shape=jax.ShapeDtypeStruct((S, D), jnp.float32),
    )(acc)


@jax.jit
def kernel(node_embeddings, batch_indices):
    idx = batch_indices.astype(jnp.int32)
    acc = _segment_sums(node_embeddings, idx)
    return _finalize(acc)


# speedup vs baseline: 5.5994x; 1.2838x over previous
import jax
import jax.numpy as jnp
from jax import lax
from jax.experimental import pallas as pl
from jax.experimental.pallas import tpu as pltpu

N = 320000
D = 128
S = 4096
B = 6400
W = 256
NB = N // B
NWIN = S // W
T = NB + 2 * NWIN
DA = D + 128


def _segment_body(bid_ref, vid_ref, fst_ref, vld_ref,
                  rows_ref, ids_ref, o_ref):
    t = pl.program_id(0)
    v = vid_ref[t]
    fstv = fst_ref[t]
    vld = vld_ref[t]

    ids = ids_ref[0, 0, :]
    local = ids - v * W
    iota = lax.broadcasted_iota(jnp.int32, (W, B), 0)
    oh = jnp.logical_and(iota == local[None, :], vld > 0)
    ohb = oh.astype(jnp.bfloat16)

    rows = rows_ref[...]
    hi = rows.astype(jnp.bfloat16)
    lo = (rows - hi.astype(jnp.float32)).astype(jnp.bfloat16)
    onecol = jnp.concatenate(
        [jnp.ones((B, 1), jnp.bfloat16), jnp.zeros((B, 127), jnp.bfloat16)],
        axis=1)
    hi_aug = jnp.concatenate([hi, onecol], axis=1)

    dn = (((1,), (0,)), ((), ()))
    contrib = lax.dot_general(ohb, hi_aug, dn,
                              preferred_element_type=jnp.float32)
    lo_c = lax.dot_general(ohb, lo, dn, preferred_element_type=jnp.float32)

    @pl.when(fstv == 1)
    def _():
        o_ref[...] = contrib
        o_ref[:, :D] += lo_c

    @pl.when(fstv == 0)
    def _():
        o_ref[...] += contrib
        o_ref[:, :D] += lo_c


def _segment_sums(x, idx):
    edges = jnp.arange(NWIN + 1, dtype=jnp.int32) * W
    ws = jnp.searchsorted(idx, edges, side="left").astype(jnp.int32)
    nonempty = ws[1:] > ws[:-1]
    fb = ws[:-1] // B
    lb = jnp.where(nonempty, (ws[1:] - 1) // B, fb)
    nb = jnp.where(nonempty, lb - fb + 1, 1)
    starts = jnp.concatenate([jnp.zeros((1,), jnp.int32),
                              jnp.cumsum(nb).astype(jnp.int32)])
    treal = starts[NWIN]

    t = jnp.arange(T, dtype=jnp.int32)
    v_t = jnp.clip(jnp.searchsorted(starts, t, side="right").astype(jnp.int32)
                   - 1, 0, NWIN - 1)
    o_t = t - starts[v_t]
    block_id = jnp.clip(jnp.where(nonempty[v_t], fb[v_t] + o_t, 0), 0, NB - 1)
    valid = (t < treal).astype(jnp.int32)
    first = jnp.logical_and(o_t == 0, t < treal).astype(jnp.int32)

    ids3 = idx.reshape(NB, 1, B)

    grid_spec = pltpu.PrefetchScalarGridSpec(
        num_scalar_prefetch=4,
        grid=(T,),
        in_specs=[
            pl.BlockSpec((B, D), lambda t, bid, vid, fst, vld: (bid[t], 0)),
            pl.BlockSpec((1, 1, B),
                         lambda t, bid, vid, fst, vld: (bid[t], 0, 0)),
        ],
        out_specs=[
            pl.BlockSpec((W, DA), lambda t, bid, vid, fst, vld: (vid[t], 0)),
        ],
    )
    (acc,) = pl.pallas_call(
        _segment_body,
        grid_spec=grid_spec,
        out_shape=[jax.ShapeDtypeStruct((S, DA), jnp.float32)],
    )(block_id, v_t, first, valid, x, ids3)
    return acc


def _divide_body(a_ref, o_ref):
    o_ref[...] = a_ref[:, :D] / jnp.maximum(a_ref[:, D:D + 1], 1.0)


def _finalize(acc):
    return pl.pallas_call(
        _divide_body,
        out_shape=jax.ShapeDtypeStruct((S, D), jnp.float32),
    )(acc)


@jax.jit
def kernel(node_embeddings, batch_indices):
    idx = batch_indices.astype(jnp.int32)
    acc = _segment_sums(node_embeddings, idx)
    return _finalize(acc)
